# Initial kernel scaffold; baseline (speedup 1.0000x reference)
import functools

import jax
import jax.numpy as jnp
from jax import lax
from jax.experimental import pallas as pl
from jax.experimental.pallas import tpu as pltpu
from jax.experimental.pallas import tpu_sc as plsc

NUM_CAM = 100000
B = 16384
NC = 2
NS = 16
NW = NC * NS
BPW = B // NW
CHUNKS = BPW // 16

_F1 = (1.0, -1.0 / 6, 1.0 / 120, -1.0 / 5040, 1.0 / 362880, -1.0 / 39916800)
_F2 = (0.5, -1.0 / 24, 1.0 / 720, -1.0 / 40320, 1.0 / 3628800, -1.0 / 479001600)


def _horner(s, coeffs):
    acc = jnp.full((16,), coeffs[-1], jnp.float32)
    for c in reversed(coeffs[:-1]):
        acc = acc * s + c
    return acc


def _sc_body(idx_hbm, table_hbm, out_hbm, idx_v, rows_v, obuf, sem):
    wid = lax.axis_index("s") * NC + lax.axis_index("c")
    base = wid * BPW

    pltpu.sync_copy(idx_hbm.at[pl.ds(wid * 4, 4), :], idx_v)

    copies = []
    for j in range(4):
        copies.append(
            pltpu.async_copy(
                table_hbm.at[idx_v.at[j]],
                rows_v.at[pl.ds(j * 128, 128), :],
                sem,
            )
        )
    for c in copies:
        c.wait()

    def chunk(c, carry):
        rid = lax.iota(jnp.int32, 16) + c * 16

        def col(k):
            return plsc.load_gather(rows_v, [rid, jnp.full((16,), k, jnp.int32)])

        tx, ty, tz = col(0), col(1), col(2)
        wx, wy, wz = col(3), col(4), col(5)
        nrms = wx * wx + wy * wy + wz * wz
        s = jnp.maximum(nrms, 1e-4)
        fac1 = _horner(s, _F1)
        fac2 = _horner(s, _F2)
        zero = jnp.zeros((16,), jnp.float32)
        vals = (
            fac2 * (wx * wx - nrms) + 1.0,
            fac2 * (wx * wy) - fac1 * wz,
            fac2 * (wx * wz) + fac1 * wy,
            tx,
            fac2 * (wy * wx) + fac1 * wz,
            fac2 * (wy * wy - nrms) + 1.0,
            fac2 * (wy * wz) - fac1 * wx,
            ty,
            fac2 * (wz * wx) - fac1 * wy,
            fac2 * (wz * wy) + fac1 * wx,
            fac2 * (wz * wz - nrms) + 1.0,
            tz,
            zero, zero, zero, zero,
        )
        for k, v in enumerate(vals):
            plsc.store_scatter(obuf, [rid, jnp.full((16,), k, jnp.int32)], v)
        return carry

    lax.fori_loop(0, CHUNKS, chunk, 0)

    pltpu.sync_copy(obuf, out_hbm.at[pl.ds(base, BPW), :])


@jax.jit
def kernel(indices, pose_adjustment):
    idx2d = indices.astype(jnp.int32).reshape(B // 128, 128)
    mesh = plsc.VectorSubcoreMesh(core_axis_name="c", subcore_axis_name="s")
    out16 = pl.kernel(
        _sc_body,
        out_type=jax.ShapeDtypeStruct((B, 16), jnp.float32),
        mesh=mesh,
        scratch_types=[
            pltpu.VMEM((4, 128), jnp.int32),
            pltpu.VMEM((BPW, 6), jnp.float32),
            pltpu.VMEM((BPW, 16), jnp.float32),
            pltpu.SemaphoreType.DMA,
        ],
    )(idx2d, pose_adjustment)
    return out16.reshape(B, 4, 4)


# speedup vs baseline: 1.1435x; 1.1435x over previous
import functools

import jax
import jax.numpy as jnp
from jax import lax
from jax.experimental import pallas as pl
from jax.experimental.pallas import tpu as pltpu
from jax.experimental.pallas import tpu_sc as plsc

NUM_CAM = 100000
B = 16384
NC = 2
NS = 16
NW = NC * NS
BPW = B // NW
CHUNKS = BPW // 16

_F1 = (1.0, -1.0 / 6, 1.0 / 120, -1.0 / 5040, 1.0 / 362880, -1.0 / 39916800)
_F2 = (0.5, -1.0 / 24, 1.0 / 720, -1.0 / 40320, 1.0 / 3628800, -1.0 / 479001600)


def _horner(s, coeffs):
    acc = jnp.full((16,), coeffs[-1], jnp.float32)
    for c in reversed(coeffs[:-1]):
        acc = acc * s + c
    return acc


def _sc_body(idx_hbm, table_hbm, out_hbm, idx_v, rows_v, obuf, sem):
    wid = lax.axis_index("s") * NC + lax.axis_index("c")
    base = wid * BPW

    pltpu.sync_copy(idx_hbm.at[pl.ds(base, BPW)], idx_v)
    pltpu.async_copy(table_hbm.at[idx_v], rows_v, sem).wait()

    def chunk(c, carry):
        rid = lax.iota(jnp.int32, 16) + c * 16

        def col(k):
            return plsc.load_gather(rows_v, [rid, jnp.full((16,), k, jnp.int32)])

        tx, ty, tz = col(0), col(1), col(2)
        wx, wy, wz = col(3), col(4), col(5)
        nrms = wx * wx + wy * wy + wz * wz
        s = jnp.maximum(nrms, 1e-4)
        fac1 = _horner(s, _F1)
        fac2 = _horner(s, _F2)
        zero = jnp.zeros((16,), jnp.float32)
        vals = (
            fac2 * (wx * wx - nrms) + 1.0,
            fac2 * (wx * wy) - fac1 * wz,
            fac2 * (wx * wz) + fac1 * wy,
            tx,
            fac2 * (wy * wx) + fac1 * wz,
            fac2 * (wy * wy - nrms) + 1.0,
            fac2 * (wy * wz) - fac1 * wx,
            ty,
            fac2 * (wz * wx) - fac1 * wy,
            fac2 * (wz * wy) + fac1 * wx,
            fac2 * (wz * wz - nrms) + 1.0,
            tz,
            zero, zero, zero, zero,
        )
        for k, v in enumerate(vals):
            plsc.store_scatter(obuf, [rid, jnp.full((16,), k, jnp.int32)], v)
        return carry

    lax.fori_loop(0, CHUNKS, chunk, 0)

    pltpu.sync_copy(obuf, out_hbm.at[pl.ds(base, BPW), :])


@jax.jit
def kernel(indices, pose_adjustment):
    idx2d = indices.astype(jnp.int32)
    table8 = jnp.pad(pose_adjustment, ((0, 0), (0, 2)))
    mesh = plsc.VectorSubcoreMesh(
        core_axis_name="c", subcore_axis_name="s", num_cores=NC, num_subcores=NS
    )
    out16 = pl.kernel(
        _sc_body,
        out_type=jax.ShapeDtypeStruct((B, 16), jnp.float32),
        mesh=mesh,
        compiler_params=pltpu.CompilerParams(
            needs_layout_passes=False, use_tc_tiling_on_sc=False
        ),
        scratch_types=[
            pltpu.VMEM((BPW,), jnp.int32),
            pltpu.VMEM((BPW, 8), jnp.float32),
            pltpu.VMEM((BPW, 16), jnp.float32),
            pltpu.SemaphoreType.DMA,
        ],
    )(idx2d, table8)
    return out16.reshape(B, 4, 4)
